# baseline (speedup 1.0000x reference)
import functools

import jax
import jax.numpy as jnp
from jax import lax
from jax.experimental import pallas as pl
from jax.experimental.pallas import tpu as pltpu
from jax.experimental.pallas import tpu_sc as plsc

N_NODES = 10000
N_PAD = 10240
N_EDGES = 320000
LANES = 16
N_WORKERS = 32
EDGE_CHUNK = 6400


def _linear_t(in_t, W, b2d, relu_in):
    d_in, n_p = in_t.shape
    d_out = W.shape[1]
    n_b = 2560

    def body(in_ref, w_ref, b_ref, out_ref):
        xb = in_ref[...]
        if relu_in:
            xb = jnp.maximum(xb, 0.0)
        acc = lax.dot_general(w_ref[...], xb, (((0,), (0,)), ((), ())),
                              preferred_element_type=jnp.float32)
        out_ref[...] = acc + b_ref[...]

    return pl.pallas_call(
        body,
        grid=(n_p // n_b,),
        in_specs=[
            pl.BlockSpec((d_in, n_b), lambda i: (0, i)),
            pl.BlockSpec((d_in, d_out), lambda i: (0, 0)),
            pl.BlockSpec((d_out, 1), lambda i: (0, 0)),
        ],
        out_specs=pl.BlockSpec((d_out, n_b), lambda i: (0, i)),
        out_shape=jax.ShapeDtypeStruct((d_out, n_p), jnp.float32),
    )(in_t, W, b2d)


def _tail(a3_t, W4, b4, W5, b5, W6, b6):
    n_p = a3_t.shape[1]
    n_b = 2560

    def body(a_ref, w4_ref, b4_ref, w5_ref, b5_ref, w6_ref, b6_ref, out_ref):
        h = jnp.maximum(a_ref[...], 0.0)
        h = lax.dot_general(w4_ref[...], h, (((0,), (0,)), ((), ())),
                            preferred_element_type=jnp.float32) + b4_ref[...]
        h = jnp.maximum(h, 0.0)
        h = lax.dot_general(w5_ref[...], h, (((0,), (0,)), ((), ())),
                            preferred_element_type=jnp.float32) + b5_ref[...]
        h = jnp.maximum(h, 0.0)
        h = lax.dot_general(w6_ref[...], h, (((0,), (0,)), ((), ())),
                            preferred_element_type=jnp.float32) + b6_ref[...]
        out_ref[...] = h

    full = lambda i: (0, 0)
    return pl.pallas_call(
        body,
        grid=(n_p // n_b,),
        in_specs=[
            pl.BlockSpec((32, n_b), lambda i: (0, i)),
            pl.BlockSpec((32, 32), full), pl.BlockSpec((32, 1), full),
            pl.BlockSpec((32, 16), full), pl.BlockSpec((16, 1), full),
            pl.BlockSpec((16, 8), full), pl.BlockSpec((8, 1), full),
        ],
        out_specs=pl.BlockSpec((8, n_b), lambda i: (0, i)),
        out_shape=jax.ShapeDtypeStruct((8, n_p), jnp.float32),
    )(a3_t, W4, b4, W5, b5, W6, b6)


@functools.lru_cache(maxsize=None)
def _make_agg(d_feat):
    f_per_w = d_feat // N_WORKERS
    n_chunks = N_EDGES // EDGE_CHUNK
    mesh = plsc.VectorSubcoreMesh(core_axis_name="c", subcore_axis_name="s")

    @functools.partial(
        pl.kernel,
        out_type=jax.ShapeDtypeStruct((d_feat, N_PAD), jnp.float32),
        mesh=mesh,
        compiler_params=pltpu.CompilerParams(needs_layout_passes=False),
        scratch_types=[
            pltpu.VMEM((f_per_w * N_PAD,), jnp.float32),
            pltpu.VMEM((f_per_w * N_PAD,), jnp.float32),
            pltpu.VMEM((EDGE_CHUNK,), jnp.int32),
            pltpu.VMEM((EDGE_CHUNK,), jnp.int32),
            pltpu.VMEM((EDGE_CHUNK,), jnp.float32),
            pltpu.VMEM((EDGE_CHUNK,), jnp.int32),
            pltpu.VMEM((EDGE_CHUNK,), jnp.int32),
            pltpu.VMEM((EDGE_CHUNK,), jnp.float32),
            pltpu.SemaphoreType.DMA,
            pltpu.SemaphoreType.DMA,
        ],
    )
    def agg(h_hbm, src_hbm, dst_hbm, w_hbm, out_hbm,
            tab_v, acc_v, src0_v, dst0_v, w0_v, src1_v, dst1_v, w1_v,
            sem0, sem1):
        cid = lax.axis_index("c")
        sid = lax.axis_index("s")
        wid = sid * 2 + cid
        base_f = wid * f_per_w

        bufs = ((src0_v, dst0_v, w0_v, sem0), (src1_v, dst1_v, w1_v, sem1))

        def issue(chunk, b):
            sv, dv, wv, sem = bufs[b]
            off = chunk * EDGE_CHUNK
            pltpu.async_copy(src_hbm.at[pl.ds(off, EDGE_CHUNK)], sv, sem)
            pltpu.async_copy(dst_hbm.at[pl.ds(off, EDGE_CHUNK)], dv, sem)
            pltpu.async_copy(w_hbm.at[pl.ds(off, EDGE_CHUNK)], wv, sem)

        def drain(b):
            sv, dv, wv, sem = bufs[b]
            z = pl.ds(0, EDGE_CHUNK)
            pltpu.make_async_copy(src_hbm.at[z], sv, sem).wait()
            pltpu.make_async_copy(dst_hbm.at[z], dv, sem).wait()
            pltpu.make_async_copy(w_hbm.at[z], wv, sem).wait()

        issue(0, 0)
        issue(1, 1)

        for f in range(f_per_w):
            pltpu.sync_copy(h_hbm.at[base_f + f],
                            tab_v.at[pl.ds(f * N_PAD, N_PAD)])

        zeros = jnp.zeros((LANES,), jnp.float32)

        @pl.loop(0, f_per_w * N_PAD // LANES, unroll=8)
        def zero_body(j):
            acc_v[pl.ds(j * LANES, LANES)] = zeros

        @pl.loop(0, n_chunks, step=2)
        def chunk_body(ci):
            for b in range(2):
                sv, dv, wv, _ = bufs[b]
                chunk = ci + b
                drain(b)

                unroll = 2 if f_per_w >= 4 else 4
                n_pend = unroll * f_per_w
                init = tuple((jnp.zeros((LANES,), jnp.int32),
                              jnp.zeros((LANES,), jnp.float32))
                             for _ in range(n_pend))

                @pl.loop(0, EDGE_CHUNK // LANES, step=unroll,
                         init_carry=init)
                def vec_body(i0, carry):
                    pend = []
                    for u in range(unroll):
                        sl = pl.ds((i0 + u) * LANES, LANES)
                        si = sv[sl]
                        di = dv[sl]
                        we = wv[sl]
                        for f in range(f_per_w):
                            g = plsc.load_gather(
                                tab_v.at[pl.ds(f * N_PAD, N_PAD)], [si])
                            pend.append((di, g * we))
                    for k, (idx, val) in enumerate(carry):
                        f = k % f_per_w
                        plsc.addupdate_scatter(
                            acc_v.at[pl.ds(f * N_PAD, N_PAD)], [idx], val)
                    return tuple(pend)

                for k, (idx, val) in enumerate(vec_body):
                    f = k % f_per_w
                    plsc.addupdate_scatter(
                        acc_v.at[pl.ds(f * N_PAD, N_PAD)], [idx], val)

                @pl.when(chunk + 2 < n_chunks)
                def _():
                    issue(chunk + 2, b)

        for f in range(f_per_w):
            pltpu.sync_copy(acc_v.at[pl.ds(f * N_PAD, N_PAD)],
                            out_hbm.at[base_f + f])

    return agg


def kernel(x, edge_index, edge_weight, W1, b1, W2, b2, W3, b3,
           W4, b4, W5, b5, W6, b6):
    src = edge_index[1].astype(jnp.int32)
    dst = edge_index[0].astype(jnp.int32)
    w = edge_weight.astype(jnp.float32)

    x_t = jnp.pad(x, ((0, N_PAD - N_NODES), (0, 0))).T
    col = lambda b: b.reshape(-1, 1)

    h1 = _linear_t(x_t, W1, col(b1), relu_in=False)
    a1 = _make_agg(128)(h1, src, dst, w)
    h2 = _linear_t(a1, W2, col(b2), relu_in=True)
    a2 = _make_agg(64)(h2, src, dst, w)
    h3 = _linear_t(a2, W3, col(b3), relu_in=True)
    a3 = _make_agg(32)(h3, src, dst, w)
    out_t = _tail(a3, W4, col(b4), W5, col(b5), W6, col(b6))
    return out_t[:, :N_NODES].T


# speedup vs baseline: 1.0656x; 1.0656x over previous
import functools

import jax
import jax.numpy as jnp
from jax import lax
from jax.experimental import pallas as pl
from jax.experimental.pallas import tpu as pltpu
from jax.experimental.pallas import tpu_sc as plsc

N_NODES = 10000
N_PAD = 10240
N_EDGES = 320000
LANES = 16
N_WORKERS = 32
EDGE_CHUNK = 6400


def _linear_t(in_t, W, b2d, relu_in):
    d_in, n_p = in_t.shape
    d_out = W.shape[1]
    n_b = 2560

    def body(in_ref, w_ref, b_ref, out_ref):
        xb = in_ref[...]
        if relu_in:
            xb = jnp.maximum(xb, 0.0)
        acc = lax.dot_general(w_ref[...], xb, (((0,), (0,)), ((), ())),
                              preferred_element_type=jnp.float32)
        out_ref[...] = acc + b_ref[...]

    return pl.pallas_call(
        body,
        grid=(n_p // n_b,),
        in_specs=[
            pl.BlockSpec((d_in, n_b), lambda i: (0, i)),
            pl.BlockSpec((d_in, d_out), lambda i: (0, 0)),
            pl.BlockSpec((d_out, 1), lambda i: (0, 0)),
        ],
        out_specs=pl.BlockSpec((d_out, n_b), lambda i: (0, i)),
        out_shape=jax.ShapeDtypeStruct((d_out, n_p), jnp.float32),
    )(in_t, W, b2d)


def _tail(a3_t, W4, b4, W5, b5, W6, b6):
    n_p = a3_t.shape[1]
    n_b = 2560

    def body(a_ref, w4_ref, b4_ref, w5_ref, b5_ref, w6_ref, b6_ref, out_ref):
        h = jnp.maximum(a_ref[...], 0.0)
        h = lax.dot_general(w4_ref[...], h, (((0,), (0,)), ((), ())),
                            preferred_element_type=jnp.float32) + b4_ref[...]
        h = jnp.maximum(h, 0.0)
        h = lax.dot_general(w5_ref[...], h, (((0,), (0,)), ((), ())),
                            preferred_element_type=jnp.float32) + b5_ref[...]
        h = jnp.maximum(h, 0.0)
        h = lax.dot_general(w6_ref[...], h, (((0,), (0,)), ((), ())),
                            preferred_element_type=jnp.float32) + b6_ref[...]
        out_ref[...] = h

    full = lambda i: (0, 0)
    return pl.pallas_call(
        body,
        grid=(n_p // n_b,),
        in_specs=[
            pl.BlockSpec((32, n_b), lambda i: (0, i)),
            pl.BlockSpec((32, 32), full), pl.BlockSpec((32, 1), full),
            pl.BlockSpec((32, 16), full), pl.BlockSpec((16, 1), full),
            pl.BlockSpec((16, 8), full), pl.BlockSpec((8, 1), full),
        ],
        out_specs=pl.BlockSpec((8, n_b), lambda i: (0, i)),
        out_shape=jax.ShapeDtypeStruct((8, n_p), jnp.float32),
    )(a3_t, W4, b4, W5, b5, W6, b6)


@functools.lru_cache(maxsize=None)
def _make_agg(d_feat):
    f_per_w = d_feat // N_WORKERS
    n_chunks = N_EDGES // EDGE_CHUNK
    mesh = plsc.VectorSubcoreMesh(core_axis_name="c", subcore_axis_name="s")

    @functools.partial(
        pl.kernel,
        out_type=jax.ShapeDtypeStruct((d_feat, N_PAD), jnp.float32),
        mesh=mesh,
        compiler_params=pltpu.CompilerParams(needs_layout_passes=False),
        scratch_types=[
            pltpu.VMEM((f_per_w * N_PAD,), jnp.float32),
            pltpu.VMEM((f_per_w * N_PAD,), jnp.float32),
            pltpu.VMEM((EDGE_CHUNK,), jnp.int32),
            pltpu.VMEM((EDGE_CHUNK,), jnp.int32),
            pltpu.VMEM((EDGE_CHUNK,), jnp.float32),
            pltpu.VMEM((EDGE_CHUNK,), jnp.int32),
            pltpu.VMEM((EDGE_CHUNK,), jnp.int32),
            pltpu.VMEM((EDGE_CHUNK,), jnp.float32),
            pltpu.SemaphoreType.DMA,
            pltpu.SemaphoreType.DMA,
        ],
    )
    def agg(h_hbm, src_hbm, dst_hbm, w_hbm, out_hbm,
            tab_v, acc_v, src0_v, dst0_v, w0_v, src1_v, dst1_v, w1_v,
            sem0, sem1):
        cid = lax.axis_index("c")
        sid = lax.axis_index("s")
        wid = sid * 2 + cid
        base_f = wid * f_per_w

        bufs = ((src0_v, dst0_v, w0_v, sem0), (src1_v, dst1_v, w1_v, sem1))

        def issue(chunk, b):
            sv, dv, wv, sem = bufs[b]
            off = chunk * EDGE_CHUNK
            pltpu.async_copy(src_hbm.at[pl.ds(off, EDGE_CHUNK)], sv, sem)
            pltpu.async_copy(dst_hbm.at[pl.ds(off, EDGE_CHUNK)], dv, sem)
            pltpu.async_copy(w_hbm.at[pl.ds(off, EDGE_CHUNK)], wv, sem)

        def drain(b):
            sv, dv, wv, sem = bufs[b]
            z = pl.ds(0, EDGE_CHUNK)
            pltpu.make_async_copy(src_hbm.at[z], sv, sem).wait()
            pltpu.make_async_copy(dst_hbm.at[z], dv, sem).wait()
            pltpu.make_async_copy(w_hbm.at[z], wv, sem).wait()

        issue(0, 0)
        issue(1, 1)

        for f in range(f_per_w):
            pltpu.sync_copy(h_hbm.at[base_f + f],
                            tab_v.at[pl.ds(f * N_PAD, N_PAD)])

        zeros = jnp.zeros((LANES,), jnp.float32)

        @pl.loop(0, f_per_w * N_PAD // LANES, unroll=8)
        def zero_body(j):
            acc_v[pl.ds(j * LANES, LANES)] = zeros

        @pl.loop(0, n_chunks, step=2)
        def chunk_body(ci):
            for b in range(2):
                sv, dv, wv, _ = bufs[b]
                chunk = ci + b
                drain(b)

                @plsc.parallel_loop(0, EDGE_CHUNK // LANES, unroll=4)
                def vec_body(i):
                    sl = pl.ds(i * LANES, LANES)
                    si = sv[sl]
                    di = dv[sl]
                    we = wv[sl]
                    for f in range(f_per_w):
                        g = plsc.load_gather(
                            tab_v.at[pl.ds(f * N_PAD, N_PAD)], [si])
                        plsc.addupdate_scatter(
                            acc_v.at[pl.ds(f * N_PAD, N_PAD)], [di], g * we)

                @pl.when(chunk + 2 < n_chunks)
                def _():
                    issue(chunk + 2, b)

        for f in range(f_per_w):
            pltpu.sync_copy(acc_v.at[pl.ds(f * N_PAD, N_PAD)],
                            out_hbm.at[base_f + f])

    return agg


def kernel(x, edge_index, edge_weight, W1, b1, W2, b2, W3, b3,
           W4, b4, W5, b5, W6, b6):
    src = edge_index[1].astype(jnp.int32)
    dst = edge_index[0].astype(jnp.int32)
    w = edge_weight.astype(jnp.float32)

    x_t = jnp.pad(x, ((0, N_PAD - N_NODES), (0, 0))).T
    col = lambda b: b.reshape(-1, 1)

    h1 = _linear_t(x_t, W1, col(b1), relu_in=False)
    a1 = _make_agg(128)(h1, src, dst, w)
    h2 = _linear_t(a1, W2, col(b2), relu_in=True)
    a2 = _make_agg(64)(h2, src, dst, w)
    h3 = _linear_t(a2, W3, col(b3), relu_in=True)
    a3 = _make_agg(32)(h3, src, dst, w)
    out_t = _tail(a3, W4, col(b4), W5, col(b5), W6, col(b6))
    return out_t[:, :N_NODES].T


# speedup vs baseline: 1.1707x; 1.0986x over previous
import functools

import jax
import jax.numpy as jnp
from jax import lax
from jax.experimental import pallas as pl
from jax.experimental.pallas import tpu as pltpu
from jax.experimental.pallas import tpu_sc as plsc

N_NODES = 10000
N_PAD = 10240
N_EDGES = 320000
LANES = 16
N_WORKERS = 32
EDGE_CHUNK = 6400
PACK_SHIFT = 14


def _linear_t(in_t, W, b2d, relu_in):
    d_in, n_p = in_t.shape
    d_out = W.shape[1]
    n_b = 2560

    def body(in_ref, w_ref, b_ref, out_ref):
        xb = in_ref[...]
        if relu_in:
            xb = jnp.maximum(xb, 0.0)
        acc = lax.dot_general(w_ref[...], xb, (((0,), (0,)), ((), ())),
                              preferred_element_type=jnp.float32)
        out_ref[...] = acc + b_ref[...]

    return pl.pallas_call(
        body,
        grid=(n_p // n_b,),
        in_specs=[
            pl.BlockSpec((d_in, n_b), lambda i: (0, i)),
            pl.BlockSpec((d_in, d_out), lambda i: (0, 0)),
            pl.BlockSpec((d_out, 1), lambda i: (0, 0)),
        ],
        out_specs=pl.BlockSpec((d_out, n_b), lambda i: (0, i)),
        out_shape=jax.ShapeDtypeStruct((d_out, n_p), jnp.float32),
    )(in_t, W, b2d)


def _tail(a3_t, W4, b4, W5, b5, W6, b6):
    n_p = a3_t.shape[1]
    n_b = 2560

    def body(a_ref, w4_ref, b4_ref, w5_ref, b5_ref, w6_ref, b6_ref, out_ref):
        h = jnp.maximum(a_ref[...], 0.0)
        h = lax.dot_general(w4_ref[...], h, (((0,), (0,)), ((), ())),
                            preferred_element_type=jnp.float32) + b4_ref[...]
        h = jnp.maximum(h, 0.0)
        h = lax.dot_general(w5_ref[...], h, (((0,), (0,)), ((), ())),
                            preferred_element_type=jnp.float32) + b5_ref[...]
        h = jnp.maximum(h, 0.0)
        h = lax.dot_general(w6_ref[...], h, (((0,), (0,)), ((), ())),
                            preferred_element_type=jnp.float32) + b6_ref[...]
        out_ref[...] = h

    full = lambda i: (0, 0)
    return pl.pallas_call(
        body,
        grid=(n_p // n_b,),
        in_specs=[
            pl.BlockSpec((32, n_b), lambda i: (0, i)),
            pl.BlockSpec((32, 32), full), pl.BlockSpec((32, 1), full),
            pl.BlockSpec((32, 16), full), pl.BlockSpec((16, 1), full),
            pl.BlockSpec((16, 8), full), pl.BlockSpec((8, 1), full),
        ],
        out_specs=pl.BlockSpec((8, n_b), lambda i: (0, i)),
        out_shape=jax.ShapeDtypeStruct((8, n_p), jnp.float32),
    )(a3_t, W4, b4, W5, b5, W6, b6)


@functools.lru_cache(maxsize=None)
def _make_agg(d_feat):
    f_per_w = d_feat // N_WORKERS
    n_chunks = N_EDGES // EDGE_CHUNK
    mesh = plsc.VectorSubcoreMesh(core_axis_name="c", subcore_axis_name="s")

    @functools.partial(
        pl.kernel,
        out_type=jax.ShapeDtypeStruct((d_feat, N_PAD), jnp.float32),
        mesh=mesh,
        compiler_params=pltpu.CompilerParams(needs_layout_passes=False),
        scratch_types=[
            pltpu.VMEM((f_per_w * N_PAD,), jnp.float32),
            pltpu.VMEM((f_per_w * N_PAD,), jnp.float32),
            pltpu.VMEM((EDGE_CHUNK,), jnp.int32),
            pltpu.VMEM((EDGE_CHUNK,), jnp.float32),
            pltpu.VMEM((EDGE_CHUNK,), jnp.int32),
            pltpu.VMEM((EDGE_CHUNK,), jnp.float32),
            pltpu.SemaphoreType.DMA,
            pltpu.SemaphoreType.DMA,
        ],
    )
    def agg(h_hbm, eidx_hbm, w_hbm, out_hbm,
            tab_v, acc_v, e0_v, w0_v, e1_v, w1_v, sem0, sem1):
        cid = lax.axis_index("c")
        sid = lax.axis_index("s")
        wid = sid * 2 + cid
        base_f = wid * f_per_w

        bufs = ((e0_v, w0_v, sem0), (e1_v, w1_v, sem1))

        def issue(chunk, b):
            ev, wv, sem = bufs[b]
            off = chunk * EDGE_CHUNK
            pltpu.async_copy(eidx_hbm.at[pl.ds(off, EDGE_CHUNK)], ev, sem)
            pltpu.async_copy(w_hbm.at[pl.ds(off, EDGE_CHUNK)], wv, sem)

        def drain(b):
            ev, wv, sem = bufs[b]
            z = pl.ds(0, EDGE_CHUNK)
            pltpu.make_async_copy(eidx_hbm.at[z], ev, sem).wait()
            pltpu.make_async_copy(w_hbm.at[z], wv, sem).wait()

        issue(0, 0)
        issue(1, 1)

        for f in range(f_per_w):
            pltpu.sync_copy(h_hbm.at[base_f + f],
                            tab_v.at[pl.ds(f * N_PAD, N_PAD)])

        zeros = jnp.zeros((LANES,), jnp.float32)

        @pl.loop(0, f_per_w * N_PAD // LANES, unroll=8)
        def zero_body(j):
            acc_v[pl.ds(j * LANES, LANES)] = zeros

        lo_mask = jnp.full((LANES,), (1 << PACK_SHIFT) - 1, jnp.int32)

        @pl.loop(0, n_chunks, step=2)
        def chunk_body(ci):
            for b in range(2):
                ev, wv, _ = bufs[b]
                chunk = ci + b
                drain(b)

                @plsc.parallel_loop(0, EDGE_CHUNK // LANES, unroll=4)
                def vec_body(i):
                    sl = pl.ds(i * LANES, LANES)
                    ei = ev[sl]
                    we = wv[sl]
                    si = lax.shift_right_logical(ei, PACK_SHIFT)
                    di = ei & lo_mask
                    for f in range(f_per_w):
                        g = plsc.load_gather(
                            tab_v.at[pl.ds(f * N_PAD, N_PAD)], [si])
                        plsc.addupdate_scatter(
                            acc_v.at[pl.ds(f * N_PAD, N_PAD)], [di], g * we)

                @pl.when(chunk + 2 < n_chunks)
                def _():
                    issue(chunk + 2, b)

        for f in range(f_per_w):
            pltpu.sync_copy(acc_v.at[pl.ds(f * N_PAD, N_PAD)],
                            out_hbm.at[base_f + f])

    return agg


def kernel(x, edge_index, edge_weight, W1, b1, W2, b2, W3, b3,
           W4, b4, W5, b5, W6, b6):
    src = edge_index[1].astype(jnp.int32)
    dst = edge_index[0].astype(jnp.int32)
    eidx = (src << PACK_SHIFT) | dst
    w = edge_weight.astype(jnp.float32)

    x_t = jnp.pad(x, ((0, N_PAD - N_NODES), (0, 0))).T
    col = lambda b: b.reshape(-1, 1)

    h1 = _linear_t(x_t, W1, col(b1), relu_in=False)
    a1 = _make_agg(128)(h1, eidx, w)
    h2 = _linear_t(a1, W2, col(b2), relu_in=True)
    a2 = _make_agg(64)(h2, eidx, w)
    h3 = _linear_t(a2, W3, col(b3), relu_in=True)
    a3 = _make_agg(32)(h3, eidx, w)
    out_t = _tail(a3, W4, col(b4), W5, col(b5), W6, col(b6))
    return out_t[:, :N_NODES].T


# speedup vs baseline: 1.2892x; 1.1011x over previous
import functools

import jax
import jax.numpy as jnp
from jax import lax
from jax.experimental import pallas as pl
from jax.experimental.pallas import tpu as pltpu
from jax.experimental.pallas import tpu_sc as plsc

N_NODES = 10000
N_PAD = 10240
N_EDGES = 320000
LANES = 16
N_WORKERS = 32
EDGE_CHUNK = 6400
PACK_SHIFT = 14


def _pack_pairs(acc):
    half = acc.shape[0] // 2
    lo = lax.bitcast_convert_type(acc[:half].astype(jnp.bfloat16),
                                  jnp.uint16).astype(jnp.uint32)
    hi = lax.bitcast_convert_type(acc[half:].astype(jnp.bfloat16),
                                  jnp.uint16).astype(jnp.uint32)
    return lax.bitcast_convert_type((hi << 16) | lo, jnp.float32)


def _linear_t(in_t, W, b2d, relu_in, pair=False):
    d_in, n_p = in_t.shape
    d_out = W.shape[1]
    d_rows = d_out // 2 if pair else d_out
    n_b = 2560

    def body(in_ref, w_ref, b_ref, out_ref):
        xb = in_ref[...]
        if relu_in:
            xb = jnp.maximum(xb, 0.0)
        acc = lax.dot_general(w_ref[...], xb, (((0,), (0,)), ((), ())),
                              preferred_element_type=jnp.float32)
        acc = acc + b_ref[...]
        out_ref[...] = _pack_pairs(acc) if pair else acc

    return pl.pallas_call(
        body,
        grid=(n_p // n_b,),
        in_specs=[
            pl.BlockSpec((d_in, n_b), lambda i: (0, i)),
            pl.BlockSpec((d_in, d_out), lambda i: (0, 0)),
            pl.BlockSpec((d_out, 1), lambda i: (0, 0)),
        ],
        out_specs=pl.BlockSpec((d_rows, n_b), lambda i: (0, i)),
        out_shape=jax.ShapeDtypeStruct((d_rows, n_p), jnp.float32),
    )(in_t, W, b2d)


def _tail(a3_t, W4, b4, W5, b5, W6, b6):
    n_p = a3_t.shape[1]
    n_b = 2560

    def body(a_ref, w4_ref, b4_ref, w5_ref, b5_ref, w6_ref, b6_ref, out_ref):
        h = jnp.maximum(a_ref[...], 0.0)
        h = lax.dot_general(w4_ref[...], h, (((0,), (0,)), ((), ())),
                            preferred_element_type=jnp.float32) + b4_ref[...]
        h = jnp.maximum(h, 0.0)
        h = lax.dot_general(w5_ref[...], h, (((0,), (0,)), ((), ())),
                            preferred_element_type=jnp.float32) + b5_ref[...]
        h = jnp.maximum(h, 0.0)
        h = lax.dot_general(w6_ref[...], h, (((0,), (0,)), ((), ())),
                            preferred_element_type=jnp.float32) + b6_ref[...]
        out_ref[...] = h

    full = lambda i: (0, 0)
    return pl.pallas_call(
        body,
        grid=(n_p // n_b,),
        in_specs=[
            pl.BlockSpec((32, n_b), lambda i: (0, i)),
            pl.BlockSpec((32, 32), full), pl.BlockSpec((32, 1), full),
            pl.BlockSpec((32, 16), full), pl.BlockSpec((16, 1), full),
            pl.BlockSpec((16, 8), full), pl.BlockSpec((8, 1), full),
        ],
        out_specs=pl.BlockSpec((8, n_b), lambda i: (0, i)),
        out_shape=jax.ShapeDtypeStruct((8, n_p), jnp.float32),
    )(a3_t, W4, b4, W5, b5, W6, b6)


@functools.lru_cache(maxsize=None)
def _make_agg(d_feat, pair=False):
    n_rows = d_feat // 2 if pair else d_feat
    f_per_w = n_rows // N_WORKERS
    a_per_w = f_per_w * (2 if pair else 1)
    n_chunks = N_EDGES // EDGE_CHUNK
    mesh = plsc.VectorSubcoreMesh(core_axis_name="c", subcore_axis_name="s")

    @functools.partial(
        pl.kernel,
        out_type=jax.ShapeDtypeStruct((d_feat, N_PAD), jnp.float32),
        mesh=mesh,
        compiler_params=pltpu.CompilerParams(needs_layout_passes=False),
        scratch_types=[
            pltpu.VMEM((f_per_w * N_PAD,), jnp.float32),
            pltpu.VMEM((a_per_w * N_PAD,), jnp.float32),
            pltpu.VMEM((EDGE_CHUNK,), jnp.int32),
            pltpu.VMEM((EDGE_CHUNK,), jnp.float32),
            pltpu.VMEM((EDGE_CHUNK,), jnp.int32),
            pltpu.VMEM((EDGE_CHUNK,), jnp.float32),
            pltpu.SemaphoreType.DMA,
            pltpu.SemaphoreType.DMA,
        ],
    )
    def agg(h_hbm, eidx_hbm, w_hbm, out_hbm,
            tab_v, acc_v, e0_v, w0_v, e1_v, w1_v, sem0, sem1):
        cid = lax.axis_index("c")
        sid = lax.axis_index("s")
        wid = sid * 2 + cid
        base_f = wid * f_per_w

        bufs = ((e0_v, w0_v, sem0), (e1_v, w1_v, sem1))

        def issue(chunk, b):
            ev, wv, sem = bufs[b]
            off = chunk * EDGE_CHUNK
            pltpu.async_copy(eidx_hbm.at[pl.ds(off, EDGE_CHUNK)], ev, sem)
            pltpu.async_copy(w_hbm.at[pl.ds(off, EDGE_CHUNK)], wv, sem)

        def drain(b):
            ev, wv, sem = bufs[b]
            z = pl.ds(0, EDGE_CHUNK)
            pltpu.make_async_copy(eidx_hbm.at[z], ev, sem).wait()
            pltpu.make_async_copy(w_hbm.at[z], wv, sem).wait()

        issue(0, 0)
        issue(1, 1)

        for f in range(f_per_w):
            pltpu.sync_copy(h_hbm.at[base_f + f],
                            tab_v.at[pl.ds(f * N_PAD, N_PAD)])

        zeros = jnp.zeros((LANES,), jnp.float32)

        @pl.loop(0, a_per_w * N_PAD // LANES, unroll=8)
        def zero_body(j):
            acc_v[pl.ds(j * LANES, LANES)] = zeros

        lo_mask = jnp.full((LANES,), (1 << PACK_SHIFT) - 1, jnp.int32)
        hi_mask = jnp.full((LANES,), 0xFFFF0000, jnp.uint32)

        @pl.loop(0, n_chunks, step=2)
        def chunk_body(ci):
            for b in range(2):
                ev, wv, _ = bufs[b]
                chunk = ci + b
                drain(b)

                @plsc.parallel_loop(0, EDGE_CHUNK // LANES, unroll=4)
                def vec_body(i):
                    sl = pl.ds(i * LANES, LANES)
                    ei = ev[sl]
                    we = wv[sl]
                    si = lax.shift_right_logical(ei, PACK_SHIFT)
                    di = ei & lo_mask
                    for f in range(f_per_w):
                        g = plsc.load_gather(
                            tab_v.at[pl.ds(f * N_PAD, N_PAD)], [si])
                        if pair:
                            u = plsc.bitcast(g, jnp.uint32)
                            glo = plsc.bitcast(u << 16, jnp.float32)
                            ghi = plsc.bitcast(u & hi_mask, jnp.float32)
                            plsc.addupdate_scatter(
                                acc_v.at[pl.ds(f * N_PAD, N_PAD)],
                                [di], glo * we)
                            plsc.addupdate_scatter(
                                acc_v.at[pl.ds((f_per_w + f) * N_PAD, N_PAD)],
                                [di], ghi * we)
                        else:
                            plsc.addupdate_scatter(
                                acc_v.at[pl.ds(f * N_PAD, N_PAD)],
                                [di], g * we)

                @pl.when(chunk + 2 < n_chunks)
                def _():
                    issue(chunk + 2, b)

        for f in range(f_per_w):
            pltpu.sync_copy(acc_v.at[pl.ds(f * N_PAD, N_PAD)],
                            out_hbm.at[base_f + f])
        if pair:
            for f in range(f_per_w):
                pltpu.sync_copy(
                    acc_v.at[pl.ds((f_per_w + f) * N_PAD, N_PAD)],
                    out_hbm.at[n_rows + base_f + f])

    return agg


def kernel(x, edge_index, edge_weight, W1, b1, W2, b2, W3, b3,
           W4, b4, W5, b5, W6, b6):
    src = edge_index[1].astype(jnp.int32)
    dst = edge_index[0].astype(jnp.int32)
    eidx = (src << PACK_SHIFT) | dst
    w = edge_weight.astype(jnp.float32)

    x_t = jnp.pad(x, ((0, N_PAD - N_NODES), (0, 0))).T
    col = lambda b: b.reshape(-1, 1)

    h1 = _linear_t(x_t, W1, col(b1), relu_in=False, pair=True)
    a1 = _make_agg(128, pair=True)(h1, eidx, w)
    h2 = _linear_t(a1, W2, col(b2), relu_in=True, pair=True)
    a2 = _make_agg(64, pair=True)(h2, eidx, w)
    h3 = _linear_t(a2, W3, col(b3), relu_in=True)
    a3 = _make_agg(32)(h3, eidx, w)
    out_t = _tail(a3, W4, col(b4), W5, col(b5), W6, col(b6))
    return out_t[:, :N_NODES].T
